# baseline (speedup 1.0000x reference)
import jax
import jax.numpy as jnp
import numpy as np
from jax import lax
from jax.experimental import pallas as pl
from jax.experimental.pallas import tpu as pltpu
from jax.experimental.pallas import tpu_sc as plsc

N = 10000
E = 320000
D = 128
H = 128
C = 16

NC = 2
NS = 16
NW = NC * NS
K = 80
EPW = E // NW
NCH = EPW // K
NB = 5
GC = NCH // NB
RP = 624
RLAST = N - (NS - 1) * RP
KS = 96
GCS = 7
NBS = 15
EPWS = NBS * GCS * KS
EPAD = NW * EPWS
NPAD = N + 16


def _mesh():
    return plsc.VectorSubcoreMesh(
        core_axis_name="c", subcore_axis_name="s",
        num_cores=NC, num_subcores=NS)


def _deg_body(src_hbm, dst_hbm, onesL_hbm, onesR_hbm, zeros_hbm, deg_hbm,
              src_idx, dst_idx, onesL_v, onesR_v,
              semS0, semS1, semS2, acc_sh):
    c = lax.axis_index("c")
    s = lax.axis_index("s")
    wid = c * NS + s
    pltpu.sync_copy(onesL_hbm, onesL_v)
    pltpu.sync_copy(onesR_hbm, onesR_v)

    @pl.when(s < NS - 1)
    def _():
        pltpu.sync_copy(zeros_hbm.at[pl.ds(0, RP)], acc_sh.at[pl.ds(s * RP, RP)])

    @pl.when(s == NS - 1)
    def _():
        pltpu.sync_copy(zeros_hbm.at[pl.ds(0, RLAST)],
                        acc_sh.at[pl.ds((NS - 1) * RP, RLAST)])

    plsc.subcore_barrier()

    SS = (semS0, semS1, semS2)

    def fire2(j, m):
        pltpu.async_copy(onesL_v, acc_sh.at[dst_idx.at[j]], SS[m], add=True)
        pltpu.async_copy(onesR_v, acc_sh.at[src_idx.at[j]], SS[m], add=True)

    def drain2(m):
        pltpu.make_async_copy(onesL_v, acc_sh.at[dst_idx.at[0]], SS[m]).wait()
        pltpu.make_async_copy(onesR_v, acc_sh.at[src_idx.at[0]], SS[m]).wait()

    def batch(b, carry):
        pltpu.sync_copy(src_hbm.at[wid, b], src_idx)
        pltpu.sync_copy(dst_hbm.at[wid, b], dst_idx)
        fire2(0, 0)
        fire2(1, 1)
        fire2(2, 2)

        def chunk3(k, carry2):
            j = 3 * k + 3
            drain2(0)
            fire2(j, 0)
            drain2(1)
            fire2(j + 1, 1)
            drain2(2)
            fire2(j + 2, 2)
            return carry2

        lax.fori_loop(0, (GC - 7) // 3, chunk3, 0)
        for j in range(GC - 4, GC):
            drain2(j % 3)
            fire2(j, j % 3)
        drain2((GC - 3) % 3)
        drain2((GC - 2) % 3)
        drain2((GC - 1) % 3)
        return carry

    lax.fori_loop(0, NB, batch, 0)
    plsc.subcore_barrier()

    @pl.when(s < NS - 1)
    def _():
        pltpu.sync_copy(acc_sh.at[pl.ds(s * RP, RP)],
                        deg_hbm.at[c, pl.ds(s * RP, RP)])

    @pl.when(s == NS - 1)
    def _():
        pltpu.sync_copy(acc_sh.at[pl.ds((NS - 1) * RP, RLAST)],
                        deg_hbm.at[c, pl.ds((NS - 1) * RP, RLAST)])


def _deg_call(src3d, dst3d, onesL, onesR, zeros128):
    f = pl.kernel(
        _deg_body,
        out_type=jax.ShapeDtypeStruct((NC, N, H), jnp.float32),
        mesh=_mesh(),
        scratch_types=(
            pltpu.VMEM((GC, K), jnp.int32),
            pltpu.VMEM((GC, K), jnp.int32),
            pltpu.VMEM((K, H), jnp.float32),
            pltpu.VMEM((K, H), jnp.float32),
            pltpu.SemaphoreType.DMA,
            pltpu.SemaphoreType.DMA,
            pltpu.SemaphoreType.DMA,
            pltpu.VMEM_SHARED((N, H), jnp.float32),
        ),
    )
    return f(src3d, dst3d, onesL, onesR, zeros128)


def _spmm_body(src_hbm, dst_hbm, h_hbm, zeros_hbm, out_hbm,
               srcA, srcB, dstA, dstB, rows0, rows1, rows2,
               semI, semG0, semG1, semG2, semS0, semS1, semS2, agg_sh):
    c = lax.axis_index("c")
    s = lax.axis_index("s")
    wid = c * NS + s
    pltpu.sync_copy(src_hbm.at[wid, 0], srcA)
    pltpu.sync_copy(dst_hbm.at[wid, 0], dstA)

    @pl.when(s < NS - 1)
    def _():
        pltpu.sync_copy(zeros_hbm.at[pl.ds(0, RP)], agg_sh.at[pl.ds(s * RP, RP)])

    @pl.when(s == NS - 1)
    def _():
        pltpu.sync_copy(zeros_hbm.at[pl.ds(0, RLAST)],
                        agg_sh.at[pl.ds((NS - 1) * RP, RLAST)])

    plsc.subcore_barrier()

    R = (rows0, rows1, rows2)
    SG = (semG0, semG1, semG2)
    SS = (semS0, semS1, semS2)

    def fire_g(si, j, m):
        pltpu.async_copy(h_hbm.at[si.at[j]], R[m], SG[m])

    def wait_g(si, m):
        pltpu.make_async_copy(h_hbm.at[si.at[0]], R[m], SG[m]).wait()

    def fire_s(di, j, m):
        pltpu.async_copy(R[m], agg_sh.at[di.at[j]], SS[m], add=True)

    def wait_s(di, m):
        pltpu.make_async_copy(R[m], agg_sh.at[di.at[0]], SS[m]).wait()

    ibufs = [(srcA, dstA), (srcB, dstB)]

    def step(j, si, di):
        if j >= 1:
            wait_s(di, (j - 1) % 3)
        if j + 2 <= GCS - 1:
            fire_g(si, j + 2, (j + 2) % 3)
        wait_g(si, j % 3)
        fire_s(di, j, j % 3)

    for b in range(NBS):
        si, di = ibufs[b % 2]
        sn, dn = ibufs[(b + 1) % 2]
        if b + 1 < NBS:
            pltpu.async_copy(src_hbm.at[wid, b + 1], sn, semI)
            pltpu.async_copy(dst_hbm.at[wid, b + 1], dn, semI)
        fire_g(si, 0, 0)
        fire_g(si, 1, 1)
        step(0, si, di)
        step(1, si, di)

        def steps(k, carry, si=si, di=di):
            j = 3 * k + 2
            wait_s(di, 1)
            fire_g(si, j + 2, 1)
            wait_g(si, 2)
            fire_s(di, j, 2)
            wait_s(di, 2)
            fire_g(si, j + 3, 2)
            wait_g(si, 0)
            fire_s(di, j + 1, 0)
            wait_s(di, 0)
            fire_g(si, j + 4, 0)
            wait_g(si, 1)
            fire_s(di, j + 2, 1)
            return carry

        lax.fori_loop(0, (GCS - 7) // 3, steps, 0)
        for j in range(GCS - 5, GCS):
            step(j, si, di)
        wait_s(di, (GCS - 1) % 3)
        if b + 1 < NBS:
            pltpu.make_async_copy(src_hbm.at[wid, b + 1], sn, semI).wait()
            pltpu.make_async_copy(dst_hbm.at[wid, b + 1], dn, semI).wait()

    plsc.subcore_barrier()

    @pl.when(s < NS - 1)
    def _():
        pltpu.sync_copy(agg_sh.at[pl.ds(s * RP, RP)],
                        out_hbm.at[c, pl.ds(s * RP, RP)])

    @pl.when(s == NS - 1)
    def _():
        pltpu.sync_copy(agg_sh.at[pl.ds((NS - 1) * RP, RLAST)],
                        out_hbm.at[c, pl.ds((NS - 1) * RP, RLAST)])


def _spmm_call(src3d, dst3d, h, zeros128):
    f = pl.kernel(
        _spmm_body,
        out_type=jax.ShapeDtypeStruct((NC, N, H), jnp.float32),
        mesh=_mesh(),
        scratch_types=(
            pltpu.VMEM((GCS, KS), jnp.int32),
            pltpu.VMEM((GCS, KS), jnp.int32),
            pltpu.VMEM((GCS, KS), jnp.int32),
            pltpu.VMEM((GCS, KS), jnp.int32),
            pltpu.VMEM((KS, H), jnp.float32),
            pltpu.VMEM((KS, H), jnp.float32),
            pltpu.VMEM((KS, H), jnp.float32),
            pltpu.SemaphoreType.DMA,
            pltpu.SemaphoreType.DMA,
            pltpu.SemaphoreType.DMA,
            pltpu.SemaphoreType.DMA,
            pltpu.SemaphoreType.DMA,
            pltpu.SemaphoreType.DMA,
            pltpu.SemaphoreType.DMA,
            pltpu.VMEM_SHARED((NPAD, H), jnp.float32),
        ),
    )
    return f(src3d, dst3d, h, zeros128)


BN = 1000
GRID = N // BN


def _tc1_body(x_ref, w1_ref, dp_ref,
              h1s_ref, innorm_ref, outnorm_ref, ideg_ref):
    ideg = dp_ref[0, :, 0:1] + dp_ref[1, :, 0:1]
    odeg = dp_ref[0, :, 64:65] + dp_ref[1, :, 64:65]
    inn = jnp.where(ideg > 0, lax.rsqrt(jnp.maximum(ideg, 1.0)), 1.0)
    onn = jnp.where(odeg > 0, lax.rsqrt(jnp.maximum(odeg, 1.0)), 1.0)
    innorm_ref[...] = inn
    outnorm_ref[...] = onn
    ideg_ref[...] = ideg
    h1s_ref[...] = jnp.dot(x_ref[...], w1_ref[...],
                           preferred_element_type=jnp.float32) * onn


def _tc1_call(x, w1, deg_p):
    return pl.pallas_call(
        _tc1_body,
        grid=(GRID,),
        in_specs=[
            pl.BlockSpec((BN, D), lambda i: (i, 0)),
            pl.BlockSpec((D, H), lambda i: (0, 0)),
            pl.BlockSpec((NC, BN, H), lambda i: (0, i, 0)),
        ],
        out_specs=[
            pl.BlockSpec((BN, H), lambda i: (i, 0)),
            pl.BlockSpec((BN, 1), lambda i: (i, 0)),
            pl.BlockSpec((BN, 1), lambda i: (i, 0)),
            pl.BlockSpec((BN, 1), lambda i: (i, 0)),
        ],
        out_shape=[
            jax.ShapeDtypeStruct((N, H), jnp.float32),
            jax.ShapeDtypeStruct((N, 1), jnp.float32),
            jax.ShapeDtypeStruct((N, 1), jnp.float32),
            jax.ShapeDtypeStruct((N, 1), jnp.float32),
        ],
    )(x, w1, deg_p)


def _tc2_body(p_ref, innorm_ref, outnorm_ref, b1_ref, w2_ref, h2s_ref):
    a = (p_ref[0] + p_ref[1]) * innorm_ref[...]
    h1 = jnp.maximum(a + b1_ref[...], 0.0)
    h2s_ref[...] = jnp.dot(h1, w2_ref[...],
                           preferred_element_type=jnp.float32) * outnorm_ref[...]


def _tc2_call(parts, innorm, outnorm, b1, w2):
    return pl.pallas_call(
        _tc2_body,
        grid=(GRID,),
        in_specs=[
            pl.BlockSpec((NC, BN, H), lambda i: (0, i, 0)),
            pl.BlockSpec((BN, 1), lambda i: (i, 0)),
            pl.BlockSpec((BN, 1), lambda i: (i, 0)),
            pl.BlockSpec((1, H), lambda i: (0, 0)),
            pl.BlockSpec((H, H), lambda i: (0, 0)),
        ],
        out_specs=pl.BlockSpec((BN, H), lambda i: (i, 0)),
        out_shape=jax.ShapeDtypeStruct((N, H), jnp.float32),
    )(parts, innorm, outnorm, b1, w2)


def _tc3_body(p_ref, innorm_ref, b2_ref, wcp_ref, bcp_ref, ideg_ref,
              s_ref, cs_ref, nc_ref):
    i = pl.program_id(0)
    a = (p_ref[0] + p_ref[1]) * innorm_ref[...]
    h2 = jnp.maximum(a + b2_ref[...], 0.0)
    logits = jnp.dot(h2, wcp_ref[...], preferred_element_type=jnp.float32) \
        + bcp_ref[...]
    m = jnp.max(logits, axis=1, keepdims=True)
    e = jnp.exp(logits - m)
    col = lax.broadcasted_iota(jnp.int32, e.shape, 1)
    e = jnp.where(col < C, e, 0.0)
    S = e / jnp.sum(e, axis=1, keepdims=True)
    s_ref[...] = S

    @pl.when(i == 0)
    def _():
        cs_ref[...] = jnp.zeros_like(cs_ref)
        nc_ref[...] = jnp.zeros_like(nc_ref)

    cs_ref[...] += jnp.sum(S, axis=0, keepdims=True)
    nc_ref[...] += jnp.sum(S * ideg_ref[...], axis=0, keepdims=True)


def _tc3_call(parts, innorm, b2, wcp, bcp, ideg):
    return pl.pallas_call(
        _tc3_body,
        grid=(GRID,),
        in_specs=[
            pl.BlockSpec((NC, BN, H), lambda i: (0, i, 0)),
            pl.BlockSpec((BN, 1), lambda i: (i, 0)),
            pl.BlockSpec((1, H), lambda i: (0, 0)),
            pl.BlockSpec((H, H), lambda i: (0, 0)),
            pl.BlockSpec((1, H), lambda i: (0, 0)),
            pl.BlockSpec((BN, 1), lambda i: (i, 0)),
        ],
        out_specs=[
            pl.BlockSpec((BN, H), lambda i: (i, 0)),
            pl.BlockSpec((1, H), lambda i: (0, 0)),
            pl.BlockSpec((1, H), lambda i: (0, 0)),
        ],
        out_shape=[
            jax.ShapeDtypeStruct((N, H), jnp.float32),
            jax.ShapeDtypeStruct((1, H), jnp.float32),
            jax.ShapeDtypeStruct((1, H), jnp.float32),
        ],
    )(parts, innorm, b2, wcp, bcp, ideg)


def _tc4_body(s_ref, asp_ref, cs_ref, nc_ref, loss_ref, t_acc):
    i = pl.program_id(0)

    @pl.when(i == 0)
    def _():
        t_acc[...] = jnp.zeros_like(t_acc)

    t_acc[...] += jnp.sum(s_ref[...] * (asp_ref[0] + asp_ref[1]),
                          axis=0, keepdims=True)

    @pl.when(i == GRID - 1)
    def _():
        t = jnp.sum(t_acc[...])
        ncv = nc_ref[...]
        csv = cs_ref[...]
        tr = t - jnp.sum(ncv * ncv) / (2.0 * E)
        spectral = -tr / (2.0 * E)
        collapse = jnp.sqrt(jnp.sum(csv * csv)) / N * np.float32(np.sqrt(C)) \
            - 1.0
        loss_ref[...] = jnp.broadcast_to(spectral + collapse, (1, 1))


def _tc4_call(s128, as_parts, cs, nc):
    return pl.pallas_call(
        _tc4_body,
        grid=(GRID,),
        in_specs=[
            pl.BlockSpec((BN, H), lambda i: (i, 0)),
            pl.BlockSpec((NC, BN, H), lambda i: (0, i, 0)),
            pl.BlockSpec((1, H), lambda i: (0, 0)),
            pl.BlockSpec((1, H), lambda i: (0, 0)),
        ],
        out_specs=pl.BlockSpec((1, 1), lambda i: (0, 0)),
        out_shape=jax.ShapeDtypeStruct((1, 1), jnp.float32),
        scratch_shapes=[pltpu.VMEM((1, H), jnp.float32)],
    )(s128, as_parts, cs, nc)


def kernel(features, edge_index, W1, b1, W2, b2, Wc, bc):
    src4d = edge_index[0].reshape(NW, NB, GC, K)
    dst4d = edge_index[1].reshape(NW, NB, GC, K)
    pad = EPAD - E
    srcp = jnp.concatenate(
        [edge_index[0], jnp.zeros((pad,), jnp.int32)]).reshape(NW, NBS, GCS, KS)
    dstp = jnp.concatenate(
        [edge_index[1], jnp.full((pad,), N, jnp.int32)]).reshape(NW, NBS, GCS, KS)
    lane = jnp.arange(H)
    onesL = jnp.broadcast_to((lane < 64).astype(jnp.float32), (K, H))
    onesR = jnp.broadcast_to((lane >= 64).astype(jnp.float32), (K, H))
    zeros128 = jnp.zeros((RLAST, H), jnp.float32)
    wcp = jnp.pad(Wc, ((0, 0), (0, H - C)))
    bcp = jnp.concatenate([bc, jnp.full((H - C,), -1e30, jnp.float32)])

    deg_p = _deg_call(src4d, dst4d, onesL, onesR, zeros128)
    h1s, innorm, outnorm, ideg = _tc1_call(features, W1, deg_p)
    agg1_p = _spmm_call(srcp, dstp, h1s, zeros128)
    h2s = _tc2_call(agg1_p, innorm, outnorm, b1.reshape(1, H), W2)
    agg2_p = _spmm_call(srcp, dstp, h2s, zeros128)
    s128, cs, nc = _tc3_call(agg2_p, innorm, b2.reshape(1, H), wcp,
                             bcp.reshape(1, H), ideg)
    as_p = _spmm_call(srcp, dstp, s128, zeros128)
    loss = _tc4_call(s128, as_p, cs, nc)
    return (loss[0, 0], s128[:, :C])


# speedup vs baseline: 1.7951x; 1.7951x over previous
import jax
import jax.numpy as jnp
import numpy as np
from jax import lax
from jax.experimental import pallas as pl
from jax.experimental.pallas import tpu as pltpu
from jax.experimental.pallas import tpu_sc as plsc

N = 10000
E = 320000
D = 128
H = 128
C = 16

NC = 2
NS = 16
NW = NC * NS
K = 80
EPW = E // NW
NCH = EPW // K
NB = 5
GC = NCH // NB
RP = 624
RLAST = N - (NS - 1) * RP
KS = 80
GCS = 25
NBS = 5
EPWS = NBS * GCS * KS
EPAD = NW * EPWS
NPAD = N + 16


def _mesh():
    return plsc.VectorSubcoreMesh(
        core_axis_name="c", subcore_axis_name="s",
        num_cores=NC, num_subcores=NS)


def _deg_body(src_hbm, dst_hbm, onesL_hbm, onesR_hbm, zeros_hbm, deg_hbm,
              src_idx, dst_idx, onesL_v, onesR_v,
              semS0, semS1, semS2, acc_sh):
    c = lax.axis_index("c")
    s = lax.axis_index("s")
    wid = c * NS + s
    pltpu.sync_copy(onesL_hbm, onesL_v)
    pltpu.sync_copy(onesR_hbm, onesR_v)

    @pl.when(s < NS - 1)
    def _():
        pltpu.sync_copy(zeros_hbm.at[pl.ds(0, RP)], acc_sh.at[pl.ds(s * RP, RP)])

    @pl.when(s == NS - 1)
    def _():
        pltpu.sync_copy(zeros_hbm.at[pl.ds(0, RLAST)],
                        acc_sh.at[pl.ds((NS - 1) * RP, RLAST)])

    plsc.subcore_barrier()

    SS = (semS0, semS1, semS2)

    def fire2(j, m):
        pltpu.async_copy(onesL_v, acc_sh.at[dst_idx.at[j]], SS[m], add=True)
        pltpu.async_copy(onesR_v, acc_sh.at[src_idx.at[j]], SS[m], add=True)

    def drain2(m):
        pltpu.make_async_copy(onesL_v, acc_sh.at[dst_idx.at[0]], SS[m]).wait()
        pltpu.make_async_copy(onesR_v, acc_sh.at[src_idx.at[0]], SS[m]).wait()

    def batch(b, carry):
        pltpu.sync_copy(src_hbm.at[wid, b], src_idx)
        pltpu.sync_copy(dst_hbm.at[wid, b], dst_idx)
        fire2(0, 0)
        fire2(1, 1)
        fire2(2, 2)

        def chunk3(k, carry2):
            j = 3 * k + 3
            drain2(0)
            fire2(j, 0)
            drain2(1)
            fire2(j + 1, 1)
            drain2(2)
            fire2(j + 2, 2)
            return carry2

        lax.fori_loop(0, (GC - 7) // 3, chunk3, 0)
        for j in range(GC - 4, GC):
            drain2(j % 3)
            fire2(j, j % 3)
        drain2((GC - 3) % 3)
        drain2((GC - 2) % 3)
        drain2((GC - 1) % 3)
        return carry

    lax.fori_loop(0, NB, batch, 0)
    plsc.subcore_barrier()

    @pl.when(s < NS - 1)
    def _():
        pltpu.sync_copy(acc_sh.at[pl.ds(s * RP, RP)],
                        deg_hbm.at[c, pl.ds(s * RP, RP)])

    @pl.when(s == NS - 1)
    def _():
        pltpu.sync_copy(acc_sh.at[pl.ds((NS - 1) * RP, RLAST)],
                        deg_hbm.at[c, pl.ds((NS - 1) * RP, RLAST)])


def _deg_call(src3d, dst3d, onesL, onesR, zeros128):
    f = pl.kernel(
        _deg_body,
        out_type=jax.ShapeDtypeStruct((NC, N, H), jnp.float32),
        mesh=_mesh(),
        scratch_types=(
            pltpu.VMEM((GC, K), jnp.int32),
            pltpu.VMEM((GC, K), jnp.int32),
            pltpu.VMEM((K, H), jnp.float32),
            pltpu.VMEM((K, H), jnp.float32),
            pltpu.SemaphoreType.DMA,
            pltpu.SemaphoreType.DMA,
            pltpu.SemaphoreType.DMA,
            pltpu.VMEM_SHARED((N, H), jnp.float32),
        ),
    )
    return f(src3d, dst3d, onesL, onesR, zeros128)


def _spmm_body(src_hbm, dst_hbm, h_hbm, zeros_hbm, out_hbm,
               srcA, srcB, dstA, dstB, rows0, rows1, rows2,
               semI, semG0, semG1, semG2, semS0, semS1, semS2, agg_sh):
    c = lax.axis_index("c")
    s = lax.axis_index("s")
    wid = c * NS + s
    pltpu.sync_copy(src_hbm.at[wid, 0], srcA)
    pltpu.sync_copy(dst_hbm.at[wid, 0], dstA)

    @pl.when(s < NS - 1)
    def _():
        pltpu.sync_copy(zeros_hbm.at[pl.ds(0, RP)], agg_sh.at[pl.ds(s * RP, RP)])

    @pl.when(s == NS - 1)
    def _():
        pltpu.sync_copy(zeros_hbm.at[pl.ds(0, RLAST)],
                        agg_sh.at[pl.ds((NS - 1) * RP, RLAST)])

    plsc.subcore_barrier()

    R = (rows0, rows1, rows2)
    SG = (semG0, semG1, semG2)
    SS = (semS0, semS1, semS2)

    def fire_g(si, j, m):
        pltpu.async_copy(h_hbm.at[si.at[j]], R[m], SG[m])

    def wait_g(si, m):
        pltpu.make_async_copy(h_hbm.at[si.at[0]], R[m], SG[m]).wait()

    def fire_s(di, j, m):
        pltpu.async_copy(R[m], agg_sh.at[di.at[j]], SS[m], add=True)

    def wait_s(di, m):
        pltpu.make_async_copy(R[m], agg_sh.at[di.at[0]], SS[m]).wait()

    ibufs = [(srcA, dstA), (srcB, dstB)]

    def step(j, si, di):
        if j >= 1:
            wait_s(di, (j - 1) % 3)
        if j + 2 <= GCS - 1:
            fire_g(si, j + 2, (j + 2) % 3)
        wait_g(si, j % 3)
        fire_s(di, j, j % 3)

    for b in range(NBS):
        si, di = ibufs[b % 2]
        sn, dn = ibufs[(b + 1) % 2]
        if b + 1 < NBS:
            pltpu.async_copy(src_hbm.at[wid, b + 1], sn, semI)
            pltpu.async_copy(dst_hbm.at[wid, b + 1], dn, semI)
        fire_g(si, 0, 0)
        fire_g(si, 1, 1)
        step(0, si, di)
        step(1, si, di)

        def steps(k, carry, si=si, di=di):
            j = 3 * k + 2
            wait_s(di, 1)
            fire_g(si, j + 2, 1)
            wait_g(si, 2)
            fire_s(di, j, 2)
            wait_s(di, 2)
            fire_g(si, j + 3, 2)
            wait_g(si, 0)
            fire_s(di, j + 1, 0)
            wait_s(di, 0)
            fire_g(si, j + 4, 0)
            wait_g(si, 1)
            fire_s(di, j + 2, 1)
            return carry

        lax.fori_loop(0, (GCS - 7) // 3, steps, 0)
        for j in range(GCS - 5, GCS):
            step(j, si, di)
        wait_s(di, (GCS - 1) % 3)
        if b + 1 < NBS:
            pltpu.make_async_copy(src_hbm.at[wid, b + 1], sn, semI).wait()
            pltpu.make_async_copy(dst_hbm.at[wid, b + 1], dn, semI).wait()

    plsc.subcore_barrier()

    @pl.when(s < NS - 1)
    def _():
        pltpu.sync_copy(agg_sh.at[pl.ds(s * RP, RP)],
                        out_hbm.at[c, pl.ds(s * RP, RP)])

    @pl.when(s == NS - 1)
    def _():
        pltpu.sync_copy(agg_sh.at[pl.ds((NS - 1) * RP, RLAST)],
                        out_hbm.at[c, pl.ds((NS - 1) * RP, RLAST)])


def _spmm_call(src3d, dst3d, h, zeros128):
    f = pl.kernel(
        _spmm_body,
        out_type=jax.ShapeDtypeStruct((NC, N, H), jnp.float32),
        mesh=_mesh(),
        scratch_types=(
            pltpu.VMEM((GCS, KS), jnp.int32),
            pltpu.VMEM((GCS, KS), jnp.int32),
            pltpu.VMEM((GCS, KS), jnp.int32),
            pltpu.VMEM((GCS, KS), jnp.int32),
            pltpu.VMEM((KS, H), jnp.float32),
            pltpu.VMEM((KS, H), jnp.float32),
            pltpu.VMEM((KS, H), jnp.float32),
            pltpu.SemaphoreType.DMA,
            pltpu.SemaphoreType.DMA,
            pltpu.SemaphoreType.DMA,
            pltpu.SemaphoreType.DMA,
            pltpu.SemaphoreType.DMA,
            pltpu.SemaphoreType.DMA,
            pltpu.SemaphoreType.DMA,
            pltpu.VMEM_SHARED((NPAD, H), jnp.float32),
        ),
    )
    return f(src3d, dst3d, h, zeros128)


BN = 1000
GRID = N // BN


def _tc1_body(x_ref, w1_ref, dp_ref,
              h1s_ref, innorm_ref, outnorm_ref, ideg_ref):
    ideg = dp_ref[0, :, 0:1] + dp_ref[1, :, 0:1]
    odeg = dp_ref[0, :, 64:65] + dp_ref[1, :, 64:65]
    inn = jnp.where(ideg > 0, lax.rsqrt(jnp.maximum(ideg, 1.0)), 1.0)
    onn = jnp.where(odeg > 0, lax.rsqrt(jnp.maximum(odeg, 1.0)), 1.0)
    innorm_ref[...] = inn
    outnorm_ref[...] = onn
    ideg_ref[...] = ideg
    h1s_ref[...] = jnp.dot(x_ref[...], w1_ref[...],
                           preferred_element_type=jnp.float32) * onn


def _tc1_call(x, w1, deg_p):
    return pl.pallas_call(
        _tc1_body,
        grid=(GRID,),
        in_specs=[
            pl.BlockSpec((BN, D), lambda i: (i, 0)),
            pl.BlockSpec((D, H), lambda i: (0, 0)),
            pl.BlockSpec((NC, BN, H), lambda i: (0, i, 0)),
        ],
        out_specs=[
            pl.BlockSpec((BN, H), lambda i: (i, 0)),
            pl.BlockSpec((BN, 1), lambda i: (i, 0)),
            pl.BlockSpec((BN, 1), lambda i: (i, 0)),
            pl.BlockSpec((BN, 1), lambda i: (i, 0)),
        ],
        out_shape=[
            jax.ShapeDtypeStruct((N, H), jnp.float32),
            jax.ShapeDtypeStruct((N, 1), jnp.float32),
            jax.ShapeDtypeStruct((N, 1), jnp.float32),
            jax.ShapeDtypeStruct((N, 1), jnp.float32),
        ],
    )(x, w1, deg_p)


def _tc2_body(p_ref, innorm_ref, outnorm_ref, b1_ref, w2_ref, h2s_ref):
    a = (p_ref[0] + p_ref[1]) * innorm_ref[...]
    h1 = jnp.maximum(a + b1_ref[...], 0.0)
    h2s_ref[...] = jnp.dot(h1, w2_ref[...],
                           preferred_element_type=jnp.float32) * outnorm_ref[...]


def _tc2_call(parts, innorm, outnorm, b1, w2):
    return pl.pallas_call(
        _tc2_body,
        grid=(GRID,),
        in_specs=[
            pl.BlockSpec((NC, BN, H), lambda i: (0, i, 0)),
            pl.BlockSpec((BN, 1), lambda i: (i, 0)),
            pl.BlockSpec((BN, 1), lambda i: (i, 0)),
            pl.BlockSpec((1, H), lambda i: (0, 0)),
            pl.BlockSpec((H, H), lambda i: (0, 0)),
        ],
        out_specs=pl.BlockSpec((BN, H), lambda i: (i, 0)),
        out_shape=jax.ShapeDtypeStruct((N, H), jnp.float32),
    )(parts, innorm, outnorm, b1, w2)


def _tc3_body(p_ref, innorm_ref, b2_ref, wcp_ref, bcp_ref, ideg_ref,
              s_ref, cs_ref, nc_ref):
    i = pl.program_id(0)
    a = (p_ref[0] + p_ref[1]) * innorm_ref[...]
    h2 = jnp.maximum(a + b2_ref[...], 0.0)
    logits = jnp.dot(h2, wcp_ref[...], preferred_element_type=jnp.float32) \
        + bcp_ref[...]
    m = jnp.max(logits, axis=1, keepdims=True)
    e = jnp.exp(logits - m)
    col = lax.broadcasted_iota(jnp.int32, e.shape, 1)
    e = jnp.where(col < C, e, 0.0)
    S = e / jnp.sum(e, axis=1, keepdims=True)
    s_ref[...] = S

    @pl.when(i == 0)
    def _():
        cs_ref[...] = jnp.zeros_like(cs_ref)
        nc_ref[...] = jnp.zeros_like(nc_ref)

    cs_ref[...] += jnp.sum(S, axis=0, keepdims=True)
    nc_ref[...] += jnp.sum(S * ideg_ref[...], axis=0, keepdims=True)


def _tc3_call(parts, innorm, b2, wcp, bcp, ideg):
    return pl.pallas_call(
        _tc3_body,
        grid=(GRID,),
        in_specs=[
            pl.BlockSpec((NC, BN, H), lambda i: (0, i, 0)),
            pl.BlockSpec((BN, 1), lambda i: (i, 0)),
            pl.BlockSpec((1, H), lambda i: (0, 0)),
            pl.BlockSpec((H, H), lambda i: (0, 0)),
            pl.BlockSpec((1, H), lambda i: (0, 0)),
            pl.BlockSpec((BN, 1), lambda i: (i, 0)),
        ],
        out_specs=[
            pl.BlockSpec((BN, H), lambda i: (i, 0)),
            pl.BlockSpec((1, H), lambda i: (0, 0)),
            pl.BlockSpec((1, H), lambda i: (0, 0)),
        ],
        out_shape=[
            jax.ShapeDtypeStruct((N, H), jnp.float32),
            jax.ShapeDtypeStruct((1, H), jnp.float32),
            jax.ShapeDtypeStruct((1, H), jnp.float32),
        ],
    )(parts, innorm, b2, wcp, bcp, ideg)


def _tc4_body(s_ref, asp_ref, cs_ref, nc_ref, loss_ref, t_acc):
    i = pl.program_id(0)

    @pl.when(i == 0)
    def _():
        t_acc[...] = jnp.zeros_like(t_acc)

    t_acc[...] += jnp.sum(s_ref[...] * (asp_ref[0] + asp_ref[1]),
                          axis=0, keepdims=True)

    @pl.when(i == GRID - 1)
    def _():
        t = jnp.sum(t_acc[...])
        ncv = nc_ref[...]
        csv = cs_ref[...]
        tr = t - jnp.sum(ncv * ncv) / (2.0 * E)
        spectral = -tr / (2.0 * E)
        collapse = jnp.sqrt(jnp.sum(csv * csv)) / N * np.float32(np.sqrt(C)) \
            - 1.0
        loss_ref[...] = jnp.broadcast_to(spectral + collapse, (1, 1))


def _tc4_call(s128, as_parts, cs, nc):
    return pl.pallas_call(
        _tc4_body,
        grid=(GRID,),
        in_specs=[
            pl.BlockSpec((BN, H), lambda i: (i, 0)),
            pl.BlockSpec((NC, BN, H), lambda i: (0, i, 0)),
            pl.BlockSpec((1, H), lambda i: (0, 0)),
            pl.BlockSpec((1, H), lambda i: (0, 0)),
        ],
        out_specs=pl.BlockSpec((1, 1), lambda i: (0, 0)),
        out_shape=jax.ShapeDtypeStruct((1, 1), jnp.float32),
        scratch_shapes=[pltpu.VMEM((1, H), jnp.float32)],
    )(s128, as_parts, cs, nc)


def kernel(features, edge_index, W1, b1, W2, b2, Wc, bc):
    src4d = edge_index[0].reshape(NW, NB, GC, K)
    dst4d = edge_index[1].reshape(NW, NB, GC, K)
    pad = EPAD - E
    srcp = jnp.concatenate(
        [edge_index[0], jnp.zeros((pad,), jnp.int32)]).reshape(NW, NBS, GCS, KS)
    dstp = jnp.concatenate(
        [edge_index[1], jnp.full((pad,), N, jnp.int32)]).reshape(NW, NBS, GCS, KS)
    lane = jnp.arange(H)
    onesL = jnp.broadcast_to((lane < 64).astype(jnp.float32), (K, H))
    onesR = jnp.broadcast_to((lane >= 64).astype(jnp.float32), (K, H))
    zeros128 = jnp.zeros((RLAST, H), jnp.float32)
    wcp = jnp.pad(Wc, ((0, 0), (0, H - C)))
    bcp = jnp.concatenate([bc, jnp.full((H - C,), -1e30, jnp.float32)])

    deg_p = _deg_call(src4d, dst4d, onesL, onesR, zeros128)
    h1s, innorm, outnorm, ideg = _tc1_call(features, W1, deg_p)
    agg1_p = _spmm_call(srcp, dstp, h1s, zeros128)
    h2s = _tc2_call(agg1_p, innorm, outnorm, b1.reshape(1, H), W2)
    agg2_p = _spmm_call(srcp, dstp, h2s, zeros128)
    s128, cs, nc = _tc3_call(agg2_p, innorm, b2.reshape(1, H), wcp,
                             bcp.reshape(1, H), ideg)
    as_p = _spmm_call(srcp, dstp, s128, zeros128)
    loss = _tc4_call(s128, as_p, cs, nc)
    return (loss[0, 0], s128[:, :C])


# speedup vs baseline: 1.8075x; 1.0069x over previous
import jax
import jax.numpy as jnp
import numpy as np
from jax import lax
from jax.experimental import pallas as pl
from jax.experimental.pallas import tpu as pltpu
from jax.experimental.pallas import tpu_sc as plsc

N = 10000
E = 320000
D = 128
H = 128
C = 16

NC = 2
NS = 16
NW = NC * NS
K = 80
EPW = E // NW
NCH = EPW // K
NB = 5
GC = NCH // NB
RP = 624
RLAST = N - (NS - 1) * RP
KS = 80
GCS = 25
NBS = 5
EPWS = NBS * GCS * KS
EPAD = NW * EPWS
NPAD = N + 16


def _mesh():
    return plsc.VectorSubcoreMesh(
        core_axis_name="c", subcore_axis_name="s",
        num_cores=NC, num_subcores=NS)


def _deg_body(src_hbm, dst_hbm, onesL_hbm, onesR_hbm, zeros_hbm, deg_hbm,
              src_idx, dst_idx, onesL_v, onesR_v,
              semS0, semS1, semS2, acc_sh):
    c = lax.axis_index("c")
    s = lax.axis_index("s")
    wid = c * NS + s
    pltpu.sync_copy(onesL_hbm, onesL_v)
    pltpu.sync_copy(onesR_hbm, onesR_v)

    @pl.when(s < NS - 1)
    def _():
        pltpu.sync_copy(zeros_hbm.at[pl.ds(0, RP)], acc_sh.at[pl.ds(s * RP, RP)])

    @pl.when(s == NS - 1)
    def _():
        pltpu.sync_copy(zeros_hbm.at[pl.ds(0, RLAST)],
                        acc_sh.at[pl.ds((NS - 1) * RP, RLAST)])

    plsc.subcore_barrier()

    SS = (semS0, semS1, semS2)

    def fire2(j, m):
        pltpu.async_copy(onesL_v, acc_sh.at[dst_idx.at[j]], SS[m], add=True)
        pltpu.async_copy(onesR_v, acc_sh.at[src_idx.at[j]], SS[m], add=True)

    def drain2(m):
        pltpu.make_async_copy(onesL_v, acc_sh.at[dst_idx.at[0]], SS[m]).wait()
        pltpu.make_async_copy(onesR_v, acc_sh.at[src_idx.at[0]], SS[m]).wait()

    def batch(b, carry):
        pltpu.sync_copy(src_hbm.at[wid, b], src_idx)
        pltpu.sync_copy(dst_hbm.at[wid, b], dst_idx)
        fire2(0, 0)
        fire2(1, 1)
        fire2(2, 2)

        def chunk3(k, carry2):
            j = 3 * k + 3
            drain2(0)
            fire2(j, 0)
            drain2(1)
            fire2(j + 1, 1)
            drain2(2)
            fire2(j + 2, 2)
            return carry2

        lax.fori_loop(0, (GC - 7) // 3, chunk3, 0)
        for j in range(GC - 4, GC):
            drain2(j % 3)
            fire2(j, j % 3)
        drain2((GC - 3) % 3)
        drain2((GC - 2) % 3)
        drain2((GC - 1) % 3)
        return carry

    lax.fori_loop(0, NB, batch, 0)
    plsc.subcore_barrier()

    @pl.when(s < NS - 1)
    def _():
        pltpu.sync_copy(acc_sh.at[pl.ds(s * RP, RP)],
                        deg_hbm.at[c, pl.ds(s * RP, RP)])

    @pl.when(s == NS - 1)
    def _():
        pltpu.sync_copy(acc_sh.at[pl.ds((NS - 1) * RP, RLAST)],
                        deg_hbm.at[c, pl.ds((NS - 1) * RP, RLAST)])


def _deg_call(src3d, dst3d, onesL, onesR, zeros128):
    f = pl.kernel(
        _deg_body,
        out_type=jax.ShapeDtypeStruct((NC, N, H), jnp.float32),
        mesh=_mesh(),
        scratch_types=(
            pltpu.VMEM((GC, K), jnp.int32),
            pltpu.VMEM((GC, K), jnp.int32),
            pltpu.VMEM((K, H), jnp.float32),
            pltpu.VMEM((K, H), jnp.float32),
            pltpu.SemaphoreType.DMA,
            pltpu.SemaphoreType.DMA,
            pltpu.SemaphoreType.DMA,
            pltpu.VMEM_SHARED((N, H), jnp.float32),
        ),
    )
    return f(src3d, dst3d, onesL, onesR, zeros128)


def _spmm_body(src_hbm, dst_hbm, h_hbm, zeros_hbm, out_hbm,
               srcA, srcB, dstA, dstB, rows0, rows1, rows2,
               semI, semG0, semG1, semG2, semS0, semS1, semS2, agg_sh):
    c = lax.axis_index("c")
    s = lax.axis_index("s")
    wid = c * NS + s
    pltpu.async_copy(src_hbm.at[wid, 0], srcA, semI)
    pltpu.async_copy(dst_hbm.at[wid, 0], dstA, semI)

    @pl.when(s < NS - 1)
    def _():
        pltpu.async_copy(zeros_hbm.at[pl.ds(0, RP)],
                         agg_sh.at[pl.ds(s * RP, RP)], semS0)
        pltpu.make_async_copy(zeros_hbm.at[pl.ds(0, RP)],
                              agg_sh.at[pl.ds(s * RP, RP)], semS0).wait()

    @pl.when(s == NS - 1)
    def _():
        pltpu.async_copy(zeros_hbm.at[pl.ds(0, RLAST)],
                         agg_sh.at[pl.ds((NS - 1) * RP, RLAST)], semS0)
        pltpu.make_async_copy(zeros_hbm.at[pl.ds(0, RLAST)],
                              agg_sh.at[pl.ds((NS - 1) * RP, RLAST)], semS0).wait()

    pltpu.make_async_copy(src_hbm.at[wid, 0], srcA, semI).wait()
    pltpu.make_async_copy(dst_hbm.at[wid, 0], dstA, semI).wait()
    plsc.subcore_barrier()

    R = (rows0, rows1, rows2)
    SG = (semG0, semG1, semG2)
    SS = (semS0, semS1, semS2)

    def fire_g(si, j, m):
        pltpu.async_copy(h_hbm.at[si.at[j]], R[m], SG[m])

    def wait_g(si, m):
        pltpu.make_async_copy(h_hbm.at[si.at[0]], R[m], SG[m]).wait()

    def fire_s(di, j, m):
        pltpu.async_copy(R[m], agg_sh.at[di.at[j]], SS[m], add=True)

    def wait_s(di, m):
        pltpu.make_async_copy(R[m], agg_sh.at[di.at[0]], SS[m]).wait()

    ibufs = [(srcA, dstA), (srcB, dstB)]

    def step(j, si, di):
        if j >= 1:
            wait_s(di, (j - 1) % 3)
        if j + 2 <= GCS - 1:
            fire_g(si, j + 2, (j + 2) % 3)
        wait_g(si, j % 3)
        fire_s(di, j, j % 3)

    for b in range(NBS):
        si, di = ibufs[b % 2]
        sn, dn = ibufs[(b + 1) % 2]
        if b + 1 < NBS:
            pltpu.async_copy(src_hbm.at[wid, b + 1], sn, semI)
            pltpu.async_copy(dst_hbm.at[wid, b + 1], dn, semI)
        fire_g(si, 0, 0)
        fire_g(si, 1, 1)
        step(0, si, di)
        step(1, si, di)

        def steps(k, carry, si=si, di=di):
            j = 3 * k + 2
            wait_s(di, 1)
            fire_g(si, j + 2, 1)
            wait_g(si, 2)
            fire_s(di, j, 2)
            wait_s(di, 2)
            fire_g(si, j + 3, 2)
            wait_g(si, 0)
            fire_s(di, j + 1, 0)
            wait_s(di, 0)
            fire_g(si, j + 4, 0)
            wait_g(si, 1)
            fire_s(di, j + 2, 1)
            return carry

        lax.fori_loop(0, (GCS - 7) // 3, steps, 0)
        for j in range(GCS - 5, GCS):
            step(j, si, di)
        wait_s(di, (GCS - 1) % 3)
        if b + 1 < NBS:
            pltpu.make_async_copy(src_hbm.at[wid, b + 1], sn, semI).wait()
            pltpu.make_async_copy(dst_hbm.at[wid, b + 1], dn, semI).wait()

    plsc.subcore_barrier()

    @pl.when(s < NS - 1)
    def _():
        pltpu.sync_copy(agg_sh.at[pl.ds(s * RP, RP)],
                        out_hbm.at[c, pl.ds(s * RP, RP)])

    @pl.when(s == NS - 1)
    def _():
        pltpu.sync_copy(agg_sh.at[pl.ds((NS - 1) * RP, RLAST)],
                        out_hbm.at[c, pl.ds((NS - 1) * RP, RLAST)])


def _spmm_call(src3d, dst3d, h, zeros128):
    f = pl.kernel(
        _spmm_body,
        out_type=jax.ShapeDtypeStruct((NC, N, H), jnp.float32),
        mesh=_mesh(),
        scratch_types=(
            pltpu.VMEM((GCS, KS), jnp.int32),
            pltpu.VMEM((GCS, KS), jnp.int32),
            pltpu.VMEM((GCS, KS), jnp.int32),
            pltpu.VMEM((GCS, KS), jnp.int32),
            pltpu.VMEM((KS, H), jnp.float32),
            pltpu.VMEM((KS, H), jnp.float32),
            pltpu.VMEM((KS, H), jnp.float32),
            pltpu.SemaphoreType.DMA,
            pltpu.SemaphoreType.DMA,
            pltpu.SemaphoreType.DMA,
            pltpu.SemaphoreType.DMA,
            pltpu.SemaphoreType.DMA,
            pltpu.SemaphoreType.DMA,
            pltpu.SemaphoreType.DMA,
            pltpu.VMEM_SHARED((NPAD, H), jnp.float32),
        ),
    )
    return f(src3d, dst3d, h, zeros128)


BN = 1000
GRID = N // BN


def _tc1_body(x_ref, w1_ref, dp_ref,
              h1s_ref, innorm_ref, outnorm_ref, ideg_ref):
    ideg = dp_ref[0, :, 0:1] + dp_ref[1, :, 0:1]
    odeg = dp_ref[0, :, 64:65] + dp_ref[1, :, 64:65]
    inn = jnp.where(ideg > 0, lax.rsqrt(jnp.maximum(ideg, 1.0)), 1.0)
    onn = jnp.where(odeg > 0, lax.rsqrt(jnp.maximum(odeg, 1.0)), 1.0)
    innorm_ref[...] = inn
    outnorm_ref[...] = onn
    ideg_ref[...] = ideg
    h1s_ref[...] = jnp.dot(x_ref[...], w1_ref[...],
                           preferred_element_type=jnp.float32) * onn


def _tc1_call(x, w1, deg_p):
    return pl.pallas_call(
        _tc1_body,
        grid=(GRID,),
        in_specs=[
            pl.BlockSpec((BN, D), lambda i: (i, 0)),
            pl.BlockSpec((D, H), lambda i: (0, 0)),
            pl.BlockSpec((NC, BN, H), lambda i: (0, i, 0)),
        ],
        out_specs=[
            pl.BlockSpec((BN, H), lambda i: (i, 0)),
            pl.BlockSpec((BN, 1), lambda i: (i, 0)),
            pl.BlockSpec((BN, 1), lambda i: (i, 0)),
            pl.BlockSpec((BN, 1), lambda i: (i, 0)),
        ],
        out_shape=[
            jax.ShapeDtypeStruct((N, H), jnp.float32),
            jax.ShapeDtypeStruct((N, 1), jnp.float32),
            jax.ShapeDtypeStruct((N, 1), jnp.float32),
            jax.ShapeDtypeStruct((N, 1), jnp.float32),
        ],
    )(x, w1, deg_p)


def _tc2_body(p_ref, innorm_ref, outnorm_ref, b1_ref, w2_ref, h2s_ref):
    a = (p_ref[0] + p_ref[1]) * innorm_ref[...]
    h1 = jnp.maximum(a + b1_ref[...], 0.0)
    h2s_ref[...] = jnp.dot(h1, w2_ref[...],
                           preferred_element_type=jnp.float32) * outnorm_ref[...]


def _tc2_call(parts, innorm, outnorm, b1, w2):
    return pl.pallas_call(
        _tc2_body,
        grid=(GRID,),
        in_specs=[
            pl.BlockSpec((NC, BN, H), lambda i: (0, i, 0)),
            pl.BlockSpec((BN, 1), lambda i: (i, 0)),
            pl.BlockSpec((BN, 1), lambda i: (i, 0)),
            pl.BlockSpec((1, H), lambda i: (0, 0)),
            pl.BlockSpec((H, H), lambda i: (0, 0)),
        ],
        out_specs=pl.BlockSpec((BN, H), lambda i: (i, 0)),
        out_shape=jax.ShapeDtypeStruct((N, H), jnp.float32),
    )(parts, innorm, outnorm, b1, w2)


def _tc3_body(p_ref, innorm_ref, b2_ref, wcp_ref, bcp_ref, ideg_ref,
              s_ref, cs_ref, nc_ref):
    i = pl.program_id(0)
    a = (p_ref[0] + p_ref[1]) * innorm_ref[...]
    h2 = jnp.maximum(a + b2_ref[...], 0.0)
    logits = jnp.dot(h2, wcp_ref[...], preferred_element_type=jnp.float32) \
        + bcp_ref[...]
    m = jnp.max(logits, axis=1, keepdims=True)
    e = jnp.exp(logits - m)
    col = lax.broadcasted_iota(jnp.int32, e.shape, 1)
    e = jnp.where(col < C, e, 0.0)
    S = e / jnp.sum(e, axis=1, keepdims=True)
    s_ref[...] = S

    @pl.when(i == 0)
    def _():
        cs_ref[...] = jnp.zeros_like(cs_ref)
        nc_ref[...] = jnp.zeros_like(nc_ref)

    cs_ref[...] += jnp.sum(S, axis=0, keepdims=True)
    nc_ref[...] += jnp.sum(S * ideg_ref[...], axis=0, keepdims=True)


def _tc3_call(parts, innorm, b2, wcp, bcp, ideg):
    return pl.pallas_call(
        _tc3_body,
        grid=(GRID,),
        in_specs=[
            pl.BlockSpec((NC, BN, H), lambda i: (0, i, 0)),
            pl.BlockSpec((BN, 1), lambda i: (i, 0)),
            pl.BlockSpec((1, H), lambda i: (0, 0)),
            pl.BlockSpec((H, H), lambda i: (0, 0)),
            pl.BlockSpec((1, H), lambda i: (0, 0)),
            pl.BlockSpec((BN, 1), lambda i: (i, 0)),
        ],
        out_specs=[
            pl.BlockSpec((BN, H), lambda i: (i, 0)),
            pl.BlockSpec((1, H), lambda i: (0, 0)),
            pl.BlockSpec((1, H), lambda i: (0, 0)),
        ],
        out_shape=[
            jax.ShapeDtypeStruct((N, H), jnp.float32),
            jax.ShapeDtypeStruct((1, H), jnp.float32),
            jax.ShapeDtypeStruct((1, H), jnp.float32),
        ],
    )(parts, innorm, b2, wcp, bcp, ideg)


def _tc4_body(s_ref, asp_ref, cs_ref, nc_ref, loss_ref, t_acc):
    i = pl.program_id(0)

    @pl.when(i == 0)
    def _():
        t_acc[...] = jnp.zeros_like(t_acc)

    t_acc[...] += jnp.sum(s_ref[...] * (asp_ref[0] + asp_ref[1]),
                          axis=0, keepdims=True)

    @pl.when(i == GRID - 1)
    def _():
        t = jnp.sum(t_acc[...])
        ncv = nc_ref[...]
        csv = cs_ref[...]
        tr = t - jnp.sum(ncv * ncv) / (2.0 * E)
        spectral = -tr / (2.0 * E)
        collapse = jnp.sqrt(jnp.sum(csv * csv)) / N * np.float32(np.sqrt(C)) \
            - 1.0
        loss_ref[...] = jnp.broadcast_to(spectral + collapse, (1, 1))


def _tc4_call(s128, as_parts, cs, nc):
    return pl.pallas_call(
        _tc4_body,
        grid=(GRID,),
        in_specs=[
            pl.BlockSpec((BN, H), lambda i: (i, 0)),
            pl.BlockSpec((NC, BN, H), lambda i: (0, i, 0)),
            pl.BlockSpec((1, H), lambda i: (0, 0)),
            pl.BlockSpec((1, H), lambda i: (0, 0)),
        ],
        out_specs=pl.BlockSpec((1, 1), lambda i: (0, 0)),
        out_shape=jax.ShapeDtypeStruct((1, 1), jnp.float32),
        scratch_shapes=[pltpu.VMEM((1, H), jnp.float32)],
    )(s128, as_parts, cs, nc)


def kernel(features, edge_index, W1, b1, W2, b2, Wc, bc):
    src4d = edge_index[0].reshape(NW, NB, GC, K)
    dst4d = edge_index[1].reshape(NW, NB, GC, K)
    pad = EPAD - E
    srcp = jnp.concatenate(
        [edge_index[0], jnp.zeros((pad,), jnp.int32)]).reshape(NW, NBS, GCS, KS)
    dstp = jnp.concatenate(
        [edge_index[1], jnp.full((pad,), N, jnp.int32)]).reshape(NW, NBS, GCS, KS)
    lane = jnp.arange(H)
    onesL = jnp.broadcast_to((lane < 64).astype(jnp.float32), (K, H))
    onesR = jnp.broadcast_to((lane >= 64).astype(jnp.float32), (K, H))
    zeros128 = jnp.zeros((RLAST, H), jnp.float32)
    wcp = jnp.pad(Wc, ((0, 0), (0, H - C)))
    bcp = jnp.concatenate([bc, jnp.full((H - C,), -1e30, jnp.float32)])

    deg_p = _deg_call(src4d, dst4d, onesL, onesR, zeros128)
    h1s, innorm, outnorm, ideg = _tc1_call(features, W1, deg_p)
    agg1_p = _spmm_call(srcp, dstp, h1s, zeros128)
    h2s = _tc2_call(agg1_p, innorm, outnorm, b1.reshape(1, H), W2)
    agg2_p = _spmm_call(srcp, dstp, h2s, zeros128)
    s128, cs, nc = _tc3_call(agg2_p, innorm, b2.reshape(1, H), wcp,
                             bcp.reshape(1, H), ideg)
    as_p = _spmm_call(srcp, dstp, s128, zeros128)
    loss = _tc4_call(s128, as_p, cs, nc)
    return (loss[0, 0], s128[:, :C])
